# baseline (speedup 1.0000x reference)
import functools

import jax
import jax.numpy as jnp
from jax import lax
from jax.experimental import pallas as pl
from jax.experimental.pallas import tpu as pltpu
from jax.experimental.pallas import tpu_sc as plsc

NB = 512
LANES = 16
PIX_PER_TILE = 65536
CHUNK = 16384
NSUB = PIX_PER_TILE // CHUNK
HIST_WORDS = LANES * 2 * NB
CLS_PER_CORE = 10
ROWS = 4


def _body(nclass, probas, labels, out,
          lbuf, pbuf, hsum, hist, tbuf, acc_v, obuf, sh, sh2,
          sp0, sp1, sl0, sl1, sf0, sf1):
    cid = lax.axis_index("c")
    sid = lax.axis_index("s")
    cbase = cid * CLS_PER_CORE
    b = sid // 4
    off = (sid % 4) * PIX_PER_TILE

    lane = lax.iota(jnp.int32, LANES)
    base0 = lane * (2 * NB)
    top = jnp.full((LANES,), 2 * NB - 1, jnp.int32)
    ones = jnp.full((LANES,), 1.0, jnp.float32)
    zeros = jnp.zeros((LANES,), jnp.float32)
    sems_p = (sp0, sp1)
    sems_l = (sl0, sl1)

    def do_class(ci, c, fb, semf):
        @plsc.parallel_loop(0, HIST_WORDS // LANES, unroll=8)
        def _(i):
            hist[pl.ds(i * LANES, LANES)] = zeros

        pltpu.async_copy(probas.at[b, c, pl.ds(off, CHUNK)], pbuf.at[0], sp0)
        pltpu.async_copy(labels.at[b, pl.ds(off, CHUNK)], lbuf.at[0], sl0)
        for sc in range(NSUB):
            cur = sc % 2
            if sc + 1 < NSUB:
                nxt = (sc + 1) % 2
                pltpu.async_copy(
                    probas.at[b, c, pl.ds(off + (sc + 1) * CHUNK, CHUNK)],
                    pbuf.at[nxt], sems_p[nxt])
                pltpu.async_copy(
                    labels.at[b, pl.ds(off + (sc + 1) * CHUNK, CHUNK)],
                    lbuf.at[nxt], sems_l[nxt])
            pltpu.make_async_copy(probas.at[b, c, pl.ds(off, CHUNK)],
                                  pbuf.at[cur], sems_p[cur]).wait()
            pltpu.make_async_copy(labels.at[b, pl.ds(off, CHUNK)],
                                  lbuf.at[cur], sems_l[cur]).wait()

            @plsc.parallel_loop(0, CHUNK // LANES, unroll=8)
            def _(i):
                p = pbuf[cur, pl.ds(i * LANES, LANES)]
                lab = lbuf[cur, pl.ds(i * LANES, LANES)]
                fg = lab == c
                kp = (p * float(NB)).astype(jnp.int32)
                idx = base0 + jnp.where(fg, top - kp, kp)
                plsc.addupdate_scatter(hist, [idx], ones)

        @plsc.parallel_loop(0, (2 * NB) // LANES, unroll=4)
        def _(j):
            a = hist[pl.ds(j * LANES, LANES)]
            for l in range(1, LANES):
                a = a + hist[pl.ds(l * 2 * NB + j * LANES, LANES)]
            hsum[fb, pl.ds(j * LANES, LANES)] = a

        pltpu.async_copy(hsum.at[fb], sh.at[ci, sid], semf)

    def class_body(ci, _):
        c = cbase + ci

        @pl.when(c < nclass)
        def _():
            @pl.when(jnp.logical_and(ci >= 2, ci % 2 == 0))
            def _():
                pltpu.make_async_copy(hsum.at[0], sh.at[0, sid], sf0).wait()

            @pl.when(jnp.logical_and(ci >= 2, ci % 2 == 1))
            def _():
                pltpu.make_async_copy(hsum.at[1], sh.at[0, sid], sf1).wait()

            @pl.when(ci % 2 == 0)
            def _():
                do_class(ci, c, 0, sf0)

            @pl.when(ci % 2 == 1)
            def _():
                do_class(ci, c, 1, sf1)

        return 0

    lax.fori_loop(0, CLS_PER_CORE, class_body, 0)
    pltpu.make_async_copy(hsum.at[0], sh.at[0, sid], sf0).wait()
    pltpu.make_async_copy(hsum.at[1], sh.at[0, sid], sf1).wait()

    plsc.subcore_barrier()

    @pl.when(jnp.logical_and(sid < CLS_PER_CORE, cbase + sid < nclass))
    def _():
        nch = LANES // ROWS
        pltpu.async_copy(sh.at[sid, pl.ds(0, ROWS)], tbuf.at[0], sp0)
        for ch in range(nch):
            cur = ch % 2
            if ch + 1 < nch:
                nxt = (ch + 1) % 2
                pltpu.async_copy(sh.at[sid, pl.ds((ch + 1) * ROWS, ROWS)],
                                 tbuf.at[nxt], sems_p[nxt])
            pltpu.make_async_copy(sh.at[sid, pl.ds(0, ROWS)],
                                  tbuf.at[cur], sems_p[cur]).wait()

            @plsc.parallel_loop(0, (2 * NB) // LANES, unroll=4)
            def _(j):
                s = pl.ds(j * LANES, LANES)
                a = tbuf[cur, 0, s]
                for r in range(1, ROWS):
                    a = a + tbuf[cur, r, s]
                if ch == 0:
                    acc_v[s] = a
                else:
                    acc_v[s] = acc_v[s] + a

        def sumbody(j, cv):
            n0v, pv = cv
            return (n0v + acc_v[pl.ds(j * LANES, LANES)],
                    pv + acc_v[pl.ds(NB + j * LANES, LANES)])

        n0v, pv = lax.fori_loop(0, NB // LANES, sumbody, (zeros, zeros),
                                unroll=4)
        N0 = jnp.sum(n0v)
        P = jnp.sum(pv)

        def scbody(j, carry):
            c0v, c1v, lossv = carry
            h0 = acc_v[pl.ds(j * LANES, LANES)]
            h1 = acc_v[pl.ds(NB + j * LANES, LANES)]
            cs0 = plsc.cumsum(h0) + c0v
            cs1 = plsc.cumsum(h1) + c1v
            C0 = N0 - cs0
            U = P + C0
            V = U + h0
            rU = 1.0 / jnp.maximum(U, 1.0)
            rV = 1.0 / jnp.maximum(V, 1.0)
            jv = jnp.zeros((LANES,), jnp.int32) + j
            ek = ((jv * LANES + lane).astype(jnp.float32) + 0.5) * (1.0 / NB)
            contrib = ek * (h1 * rU + (cs1 - h1) * (rU - rV))
            return (c0v + jnp.sum(h0), c1v + jnp.sum(h1), lossv + contrib)

        _, _, lossv = lax.fori_loop(0, NB // LANES, scbody,
                                    (zeros, zeros, zeros), unroll=2)
        loss_c = jnp.sum(lossv)
        pv16 = zeros + P
        presv = jnp.where(pv16 > 0.0, ones, zeros)
        resv = jnp.where(lane == 0, zeros + loss_c,
                         jnp.where(lane == 1, presv, zeros))
        obuf[...] = resv
        pltpu.sync_copy(obuf, sh2.at[sid])

    plsc.subcore_barrier()

    @pl.when(sid == 0)
    def _():
        def fin(i, av):
            pltpu.sync_copy(sh2.at[i], obuf)
            civ = jnp.zeros((LANES,), jnp.int32) + (cbase + i)
            return av + jnp.where(civ < nclass, obuf[...], zeros)

        accv = lax.fori_loop(0, CLS_PER_CORE, fin, zeros)
        obuf[...] = accv
        pltpu.sync_copy(obuf, out.at[cid])


def kernel(input, target):
    B, C, H, W = input.shape
    HW = H * W
    probas = input.reshape(B, C, HW)
    labels = target.reshape(B, HW).astype(jnp.int32)

    mesh = plsc.VectorSubcoreMesh(core_axis_name="c", subcore_axis_name="s")
    params = pltpu.CompilerParams(needs_layout_passes=False)

    out2 = pl.kernel(
        functools.partial(_body, C),
        out_type=jax.ShapeDtypeStruct((2, LANES), jnp.float32),
        mesh=mesh,
        compiler_params=params,
        scratch_types=[
            pltpu.VMEM((2, CHUNK), jnp.int32),
            pltpu.VMEM((2, CHUNK), jnp.float32),
            pltpu.VMEM((2, 2 * NB), jnp.float32),
            pltpu.VMEM((HIST_WORDS,), jnp.float32),
            pltpu.VMEM((2, ROWS, 2 * NB), jnp.float32),
            pltpu.VMEM((2 * NB,), jnp.float32),
            pltpu.VMEM((LANES,), jnp.float32),
            pltpu.VMEM_SHARED((CLS_PER_CORE, LANES, 2 * NB), jnp.float32),
            pltpu.VMEM_SHARED((LANES, LANES), jnp.float32),
            pltpu.SemaphoreType.DMA,
            pltpu.SemaphoreType.DMA,
            pltpu.SemaphoreType.DMA,
            pltpu.SemaphoreType.DMA,
            pltpu.SemaphoreType.DMA,
            pltpu.SemaphoreType.DMA,
        ],
    )(probas, labels)

    return (out2[0, 0] + out2[1, 0]) / (out2[0, 1] + out2[1, 1])


# speedup vs baseline: 1.1012x; 1.1012x over previous
import functools

import jax
import jax.numpy as jnp
from jax import lax
from jax.experimental import pallas as pl
from jax.experimental.pallas import tpu as pltpu
from jax.experimental.pallas import tpu_sc as plsc

NB = 256
LANES = 16
NW = 32
PIX_PER_TILE = 32768
HIST_WORDS = LANES * 2 * NB


def _hist_body(nclass, probas, labels, out, labels_v, pbuf, hsum, hist,
               sp0, sp1, sf0, sf1):
    cid = lax.axis_index("c")
    sid = lax.axis_index("s")
    wid = sid * 2 + cid
    b = wid // 8
    hw0 = (wid % 8) * PIX_PER_TILE
    pltpu.sync_copy(labels.at[b, pl.ds(hw0, PIX_PER_TILE)], labels_v)

    lane = lax.iota(jnp.int32, LANES)
    base0 = lane * (2 * NB)
    top = jnp.full((LANES,), 2 * NB - 1, jnp.int32)
    ones = jnp.full((LANES,), 1.0, jnp.float32)
    zeros = jnp.zeros((LANES,), jnp.float32)
    sems_p = (sp0, sp1)

    def do_class(c, fb, semf):
        @plsc.parallel_loop(0, HIST_WORDS // LANES, unroll=8)
        def _(i):
            hist[pl.ds(i * LANES, LANES)] = zeros

        @pl.when(c + 1 < nclass)
        def _():
            pltpu.async_copy(probas.at[b, c + 1, pl.ds(hw0, PIX_PER_TILE)],
                             pbuf.at[1 - fb], sems_p[1 - fb])

        pltpu.make_async_copy(probas.at[b, c, pl.ds(hw0, PIX_PER_TILE)],
                              pbuf.at[fb], sems_p[fb]).wait()

        @plsc.parallel_loop(0, PIX_PER_TILE // LANES, unroll=8)
        def _(i):
            p = pbuf[fb, pl.ds(i * LANES, LANES)]
            lab = labels_v[pl.ds(i * LANES, LANES)]
            fg = lab == c
            kp = (p * float(NB)).astype(jnp.int32)
            idx = base0 + jnp.where(fg, top - kp, kp)
            plsc.addupdate_scatter(hist, [idx], ones)

        @plsc.parallel_loop(0, (2 * NB) // LANES, unroll=4)
        def _(j):
            acc = hist[pl.ds(j * LANES, LANES)]
            for l in range(1, LANES):
                acc = acc + hist[pl.ds(l * 2 * NB + j * LANES, LANES)]
            hsum[fb, pl.ds(j * LANES, LANES)] = acc
        pltpu.async_copy(hsum.at[fb], out.at[c, wid], semf)

    npairs = (nclass + 1) // 2
    pltpu.async_copy(probas.at[b, 0, pl.ds(hw0, PIX_PER_TILE)], pbuf.at[0],
                     sp0)

    def pair_body(g, _):
        @pl.when(g >= 1)
        def _():
            pltpu.make_async_copy(hsum.at[0], out.at[0, wid], sf0).wait()
            pltpu.make_async_copy(hsum.at[1], out.at[0, wid], sf1).wait()

        c0 = 2 * g
        do_class(c0, 0, sf0)

        @pl.when(c0 + 1 < nclass)
        def _():
            do_class(c0 + 1, 1, sf1)

        return 0

    lax.fori_loop(0, npairs, pair_body, 0)
    pltpu.make_async_copy(hsum.at[0], out.at[0, wid], sf0).wait()
    if nclass % 2 == 0:
        pltpu.make_async_copy(hsum.at[1], out.at[0, wid], sf1).wait()


ROWS = 8


def _loss_body(nclass, partials, out, acc_v, tbuf, obuf, sh, st0, st1):
    cid = lax.axis_index("c")
    sid = lax.axis_index("s")
    lane = lax.iota(jnp.int32, LANES)
    zeros = jnp.zeros((LANES,), jnp.float32)
    ones = jnp.full((LANES,), 1.0, jnp.float32)
    sems = (st0, st1)
    nch = NW // ROWS

    def do_class(cls):
        pltpu.async_copy(partials.at[cls, pl.ds(0, ROWS)], tbuf.at[0], st0)
        for ch in range(nch):
            cur = ch % 2
            if ch + 1 < nch:
                nxt = (ch + 1) % 2
                pltpu.async_copy(
                    partials.at[cls, pl.ds((ch + 1) * ROWS, ROWS)],
                    tbuf.at[nxt], sems[nxt])
            pltpu.make_async_copy(partials.at[cls, pl.ds(0, ROWS)],
                                  tbuf.at[cur], sems[cur]).wait()

            @plsc.parallel_loop(0, (2 * NB) // LANES, unroll=4)
            def _(j):
                s = pl.ds(j * LANES, LANES)
                acc = tbuf[cur, 0, s]
                for r in range(1, ROWS):
                    acc = acc + tbuf[cur, r, s]
                if ch == 0:
                    acc_v[s] = acc
                else:
                    acc_v[s] = acc_v[s] + acc

        def sumbody(j, cv):
            n0v, pv = cv
            return (n0v + acc_v[pl.ds(j * LANES, LANES)],
                    pv + acc_v[pl.ds(NB + j * LANES, LANES)])

        n0v, pv = lax.fori_loop(0, NB // LANES, sumbody, (zeros, zeros),
                                unroll=4)
        N0 = jnp.sum(n0v)
        P = jnp.sum(pv)

        def scbody(j, carry):
            c0v, c1v, lossv = carry
            h0 = acc_v[pl.ds(j * LANES, LANES)]
            h1 = acc_v[pl.ds(NB + j * LANES, LANES)]
            cs0 = plsc.cumsum(h0) + c0v
            cs1 = plsc.cumsum(h1) + c1v
            C0 = N0 - cs0
            U = P + C0
            V = U + h0
            rU = 1.0 / jnp.maximum(U, 1.0)
            rV = 1.0 / jnp.maximum(V, 1.0)
            jv = jnp.zeros((LANES,), jnp.int32) + j
            ek = ((jv * LANES + lane).astype(jnp.float32) + 0.5) * (1.0 / NB)
            contrib = ek * (h1 * rU + (cs1 - h1) * (rU - rV))
            return (c0v + jnp.sum(h0), c1v + jnp.sum(h1), lossv + contrib)

        _, _, lossv = lax.fori_loop(0, NB // LANES, scbody,
                                    (zeros, zeros, zeros), unroll=2)
        loss_c = jnp.sum(lossv)
        pv16 = zeros + P
        presv = jnp.where(pv16 > 0.0, ones, zeros)
        resv = jnp.where(lane == 0, zeros + loss_c,
                         jnp.where(lane == 1, presv, zeros))
        obuf[...] = resv
        pltpu.sync_copy(obuf, sh.at[cls])

    do_class(sid)

    @pl.when(sid < nclass - LANES)
    def _():
        do_class(sid + LANES)

    plsc.subcore_barrier()

    @pl.when(jnp.logical_and(cid == 0, sid == 0))
    def _():
        def fin(i, av):
            pltpu.sync_copy(sh.at[i], obuf)
            return av + obuf[...]

        accv = lax.fori_loop(0, nclass, fin, zeros)
        lossS = jnp.sum(jnp.where(lane == 0, accv, zeros))
        presS = jnp.sum(jnp.where(lane == 1, accv, zeros))
        obuf[...] = (zeros + lossS) / (zeros + presS)
        pltpu.sync_copy(obuf, out)


def kernel(input, target):
    B, C, H, W = input.shape
    HW = H * W
    probas = input.reshape(B, C, HW)
    labels = target.reshape(B, HW).astype(jnp.int32)

    mesh = plsc.VectorSubcoreMesh(core_axis_name="c", subcore_axis_name="s")
    params = pltpu.CompilerParams(needs_layout_passes=False)

    partials = pl.kernel(
        functools.partial(_hist_body, C),
        out_type=jax.ShapeDtypeStruct((C, NW, 2 * NB), jnp.float32),
        mesh=mesh,
        compiler_params=params,
        scratch_types=[
            pltpu.VMEM((PIX_PER_TILE,), jnp.int32),
            pltpu.VMEM((2, PIX_PER_TILE), jnp.float32),
            pltpu.VMEM((2, 2 * NB), jnp.float32),
            pltpu.VMEM((HIST_WORDS,), jnp.float32),
            pltpu.SemaphoreType.DMA,
            pltpu.SemaphoreType.DMA,
            pltpu.SemaphoreType.DMA,
            pltpu.SemaphoreType.DMA,
        ],
    )(probas, labels)

    out16 = pl.kernel(
        functools.partial(_loss_body, C),
        out_type=jax.ShapeDtypeStruct((LANES,), jnp.float32),
        mesh=mesh,
        compiler_params=params,
        scratch_types=[
            pltpu.VMEM((2 * NB,), jnp.float32),
            pltpu.VMEM((2, ROWS, 2 * NB), jnp.float32),
            pltpu.VMEM((LANES,), jnp.float32),
            pltpu.VMEM_SHARED((C, LANES), jnp.float32),
            pltpu.SemaphoreType.DMA,
            pltpu.SemaphoreType.DMA,
        ],
    )(partials)

    return out16[0]
